# Initial kernel scaffold; baseline (speedup 1.0000x reference)
import functools

import jax
import jax.numpy as jnp
from jax import lax
from jax.experimental import pallas as pl
from jax.experimental.pallas import tpu as pltpu
from jax.experimental.pallas import tpu_sc as plsc

N = 10000
E = 160000
NC = 2
NS = 16
R = 10016
RP = R // NS
EPT = E // NS
NB = 80
NT = NB // 2
EPW = E // (NC * NS)
NBD = 40

_mesh = plsc.VectorSubcoreMesh(
    core_axis_name="c", subcore_axis_name="s", num_cores=NC, num_subcores=NS)


@functools.partial(
    pl.kernel,
    out_type=jax.ShapeDtypeStruct((NC * R, 16), jnp.float32),
    mesh=_mesh,
    scratch_types=[
        pltpu.VMEM((NBD, 128), jnp.int32),
        pltpu.VMEM((128, 16), jnp.float32),
        pltpu.VMEM((RP, 16), jnp.float32),
        pltpu.VMEM_SHARED((R, 16), jnp.float32),
    ],
)
def _deg_kernel(dst_hbm, out_hbm, idx_v, ones_v, zero_v, acc):
  c = lax.axis_index("c")
  s = lax.axis_index("s")
  w = s * NC + c

  def fill_ones(i, _):
    ones_v[i] = jnp.ones((16,), jnp.float32)
    return 0
  lax.fori_loop(0, 128, fill_ones, 0)

  def fill_zero(i, _):
    zero_v[i] = jnp.zeros((16,), jnp.float32)
    return 0
  lax.fori_loop(0, RP, fill_zero, 0)

  pltpu.sync_copy(dst_hbm.at[w], idx_v)
  pltpu.sync_copy(zero_v, acc.at[pl.ds(s * RP, RP)])
  plsc.subcore_barrier()

  def body(b, _):
    pltpu.sync_copy(ones_v, acc.at[idx_v.at[b]], add=True)
    return 0
  lax.fori_loop(0, NBD, body, 0)

  plsc.subcore_barrier()
  pltpu.sync_copy(acc.at[pl.ds(s * RP, RP)],
                  out_hbm.at[pl.ds(c * R + s * RP, RP)])


def _make_agg_kernel(nch):
  cps = nch // NC

  @functools.partial(
      pl.kernel,
      out_type=jax.ShapeDtypeStruct((nch * R, 128), jnp.float32),
      mesh=_mesh,
      scratch_types=[
          pltpu.VMEM((NB, 128), jnp.int32),
          pltpu.VMEM((NB, 128), jnp.int32),
          pltpu.VMEM((2, 128, 128), jnp.float32),
          pltpu.VMEM_SHARED((R, 128), jnp.float32),
          pltpu.SemaphoreType.DMA,
          pltpu.SemaphoreType.DMA,
      ],
  )
  def agg(vals_hbm, src_hbm, dst_hbm, out_hbm,
          idx_s, idx_d, rows, acc, sem0, sem1):
    c = lax.axis_index("c")
    s = lax.axis_index("s")
    pltpu.sync_copy(dst_hbm.at[s], idx_d)

    for j in range(cps):
      chunk = c * cps + j
      base = chunk * R
      pltpu.sync_copy(vals_hbm.at[pl.ds(base + s * RP, RP)],
                      acc.at[pl.ds(s * RP, RP)])
      pltpu.sync_copy(src_hbm.at[chunk, s], idx_s)
      plsc.subcore_barrier()

      pltpu.async_copy(vals_hbm.at[idx_s.at[0]], rows.at[0], sem0)

      def body(t, _):
        b0 = 2 * t
        b1 = b0 + 1
        pltpu.async_copy(vals_hbm.at[idx_s.at[b1]], rows.at[1], sem1)
        pltpu.make_async_copy(
            vals_hbm.at[idx_s.at[b0]], rows.at[0], sem0).wait()
        pltpu.sync_copy(rows.at[0], acc.at[idx_d.at[b0]], add=True)

        @pl.when(t + 1 < NT)
        def _():
          pltpu.async_copy(vals_hbm.at[idx_s.at[b0 + 2]], rows.at[0], sem0)

        pltpu.make_async_copy(
            vals_hbm.at[idx_s.at[b1]], rows.at[1], sem1).wait()
        pltpu.sync_copy(rows.at[1], acc.at[idx_d.at[b1]], add=True)
        return 0

      lax.fori_loop(0, NT, body, 0)

      plsc.subcore_barrier()
      pltpu.sync_copy(acc.at[pl.ds(s * RP, RP)],
                      out_hbm.at[pl.ds(base + s * RP, RP)])
      plsc.subcore_barrier()

  return agg


_agg2 = _make_agg_kernel(2)
_agg4 = _make_agg_kernel(4)


def _dis_body(p0, p1, o):
  deg = p0[:, :1] + p1[:, :1] + 1.0
  o[...] = lax.rsqrt(jnp.maximum(deg, 1.0))


def _dis(p0, p1):
  return pl.pallas_call(
      _dis_body,
      out_shape=jax.ShapeDtypeStruct((R, 1), jnp.float32),
  )(p0, p1)


def _prescale_body(x, d, o):
  o[0] = x[...] * d[...]


def _prescale(x, dis):
  mb = 400
  return pl.pallas_call(
      _prescale_body,
      grid=(N // mb, 2),
      in_specs=[
          pl.BlockSpec((mb, 128), lambda i, c: (i, c)),
          pl.BlockSpec((mb, 1), lambda i, c: (i, 0)),
      ],
      out_specs=pl.BlockSpec((1, mb, 128), lambda i, c: (c, i, 0)),
      out_shape=jax.ShapeDtypeStruct((2, R, 128), jnp.float32),
  )(x, dis)


def _mm1_body(a, w, d, b, o, acc):
  k = pl.program_id(2)

  @pl.when(k == 0)
  def _():
    acc[...] = jnp.zeros_like(acc)

  acc[...] += jnp.dot(a[0] * d[...], w[0],
                      preferred_element_type=jnp.float32)

  @pl.when(k == 1)
  def _():
    o[...] = jnp.maximum(acc[...] + b[...], 0.0)


def _mm1(agg1, w1r, dis, b1):
  mb, nb = 400, 256
  return pl.pallas_call(
      _mm1_body,
      grid=(N // mb, 1024 // nb, 2),
      in_specs=[
          pl.BlockSpec((1, mb, 128), lambda i, j, k: (k, i, 0)),
          pl.BlockSpec((1, 128, nb), lambda i, j, k: (k, 0, j)),
          pl.BlockSpec((mb, 1), lambda i, j, k: (i, 0)),
          pl.BlockSpec((1, nb), lambda i, j, k: (0, j)),
      ],
      out_specs=pl.BlockSpec((mb, nb), lambda i, j, k: (i, j)),
      out_shape=jax.ShapeDtypeStruct((N, 1024), jnp.float32),
      scratch_shapes=[pltpu.VMEM((mb, nb), jnp.float32)],
  )(agg1, w1r, dis, b1)


def _mm2_body(a, w, d, o):
  o[0] = jnp.dot(a[...], w[...],
                 preferred_element_type=jnp.float32) * d[...]


def _mm2(h, w2, dis):
  mb = 400
  return pl.pallas_call(
      _mm2_body,
      grid=(N // mb, 4),
      in_specs=[
          pl.BlockSpec((mb, 1024), lambda i, j: (i, 0)),
          pl.BlockSpec((1024, 128), lambda i, j: (0, j)),
          pl.BlockSpec((mb, 1), lambda i, j: (i, 0)),
      ],
      out_specs=pl.BlockSpec((1, mb, 128), lambda i, j: (j, i, 0)),
      out_shape=jax.ShapeDtypeStruct((4, R, 128), jnp.float32),
  )(h, w2, dis)


def _final_body(a, d, b, o):
  o[...] = a[0] * d[...] + b[...]


def _final(agg2, dis, b2r):
  mb = 400
  return pl.pallas_call(
      _final_body,
      grid=(N // mb, 4),
      in_specs=[
          pl.BlockSpec((1, mb, 128), lambda i, c: (c, i, 0)),
          pl.BlockSpec((mb, 1), lambda i, c: (i, 0)),
          pl.BlockSpec((1, 128), lambda i, c: (c, 0)),
      ],
      out_specs=pl.BlockSpec((mb, 128), lambda i, c: (i, c)),
      out_shape=jax.ShapeDtypeStruct((N, 512), jnp.float32),
  )(agg2, dis, b2r)


def kernel(x, edge_index, W1, b1, W2, b2):
  src = edge_index[0].astype(jnp.int32)
  dst = edge_index[1].astype(jnp.int32)

  src_t = jnp.pad(src.reshape(NS, EPT), ((0, 0), (0, NB * 128 - EPT)))
  src_t = src_t.reshape(NS, NB, 128)
  dst_t = jnp.pad(dst.reshape(NS, EPT), ((0, 0), (0, NB * 128 - EPT)),
                  constant_values=N).reshape(NS, NB, 128)
  off2 = (R * jnp.arange(2, dtype=jnp.int32))[:, None, None, None]
  off4 = (R * jnp.arange(4, dtype=jnp.int32))[:, None, None, None]
  src2 = src_t[None] + off2
  src4 = src_t[None] + off4

  dst_d = jnp.pad(dst.reshape(NC * NS, EPW),
                  ((0, 0), (0, NBD * 128 - EPW)),
                  constant_values=N).reshape(NC * NS, NBD, 128)

  parts = _deg_kernel(dst_d).reshape(NC, R, 16)
  dis = _dis(parts[0], parts[1])

  vals1 = _prescale(x, dis).reshape(2 * R, 128)
  agg1 = _agg2(vals1, src2, dst_t)
  h = _mm1(agg1.reshape(2, R, 128), W1.reshape(2, 128, 1024), dis,
           b1.reshape(1, 1024))
  vals2 = _mm2(h, W2, dis).reshape(4 * R, 128)
  agg2 = _agg4(vals2, src4, dst_t)
  return _final(agg2.reshape(4, R, 128), dis, b2.reshape(4, 128))


# speedup vs baseline: 6.4705x; 6.4705x over previous
import functools

import jax
import jax.numpy as jnp
from jax import lax
from jax.experimental import pallas as pl
from jax.experimental.pallas import tpu as pltpu
from jax.experimental.pallas import tpu_sc as plsc

N = 10000
E = 160000
NC = 2
NS = 16
R = 10240
RP = R // NS
EPT = E // NS
NB = 80
NBH = NB // 2
NT = NBH // 2
EPW = E // (NC * NS)
NBD = 40

_mesh = plsc.VectorSubcoreMesh(
    core_axis_name="c", subcore_axis_name="s", num_cores=NC, num_subcores=NS)


@functools.partial(
    pl.kernel,
    out_type=jax.ShapeDtypeStruct((NC * R, 128), jnp.float32),
    mesh=_mesh,
    scratch_types=[
        pltpu.VMEM((NBD, 128), jnp.int32),
        pltpu.VMEM((RP // 128, 128), jnp.int32),
        pltpu.VMEM((128, 128), jnp.float32),
        pltpu.VMEM((128, 128), jnp.float32),
        pltpu.VMEM_SHARED((R, 128), jnp.float32),
    ],
)
def _deg_kernel(dst_hbm, seq_hbm, ones_hbm, zeros_hbm, out_hbm,
                idx_v, seq_v, ones_v, stage_v, acc):
  c = lax.axis_index("c")
  s = lax.axis_index("s")
  w = s * NC + c

  pltpu.sync_copy(ones_hbm, ones_v)
  pltpu.sync_copy(zeros_hbm, stage_v)
  pltpu.sync_copy(dst_hbm.at[w], idx_v)
  pltpu.sync_copy(seq_hbm.at[s], seq_v)
  for q in range(RP // 128):
    pltpu.sync_copy(stage_v, acc.at[seq_v.at[q]])
  plsc.subcore_barrier()

  def body(b, _):
    pltpu.sync_copy(ones_v, acc.at[idx_v.at[b]], add=True)
    return 0
  lax.fori_loop(0, NBD, body, 0)

  plsc.subcore_barrier()
  for q in range(RP // 128):
    pltpu.sync_copy(acc.at[seq_v.at[q]], stage_v)
    pltpu.sync_copy(stage_v,
                    out_hbm.at[pl.ds(c * R + s * RP + q * 128, 128)])


def _make_agg_kernel(nch):
  cps = nch // NC

  @functools.partial(
      pl.kernel,
      out_type=jax.ShapeDtypeStruct((nch * R, 128), jnp.float32),
      mesh=_mesh,
      scratch_types=[
          pltpu.VMEM((NBH, 128), jnp.int32),
          pltpu.VMEM((NBH, 128), jnp.int32),
          pltpu.VMEM((RP // 128, 128), jnp.int32),
          pltpu.VMEM((2, 128, 128), jnp.float32),
          pltpu.VMEM_SHARED((R, 128), jnp.float32),
          pltpu.SemaphoreType.DMA,
          pltpu.SemaphoreType.DMA,
      ],
  )
  def agg(vals_hbm, src_hbm, dst_hbm, seq_hbm, out_hbm,
          idx_s, idx_d, seq_v, rows, acc, sem0, sem1):
    c = lax.axis_index("c")
    s = lax.axis_index("s")
    pltpu.sync_copy(seq_hbm.at[s], seq_v)

    for j in range(cps):
      chunk = c * cps + j
      base = chunk * R
      for q in range(RP // 128):
        pltpu.sync_copy(vals_hbm.at[pl.ds(base + s * RP + q * 128, 128)],
                        rows.at[0])
        pltpu.sync_copy(rows.at[0], acc.at[seq_v.at[q]])
      plsc.subcore_barrier()

      for half in range(2):
        pltpu.sync_copy(src_hbm.at[chunk, s, pl.ds(half * NBH, NBH)], idx_s)
        pltpu.sync_copy(dst_hbm.at[s, pl.ds(half * NBH, NBH)], idx_d)

        def body(t, _):
          b0 = 2 * t
          b1 = b0 + 1
          d0 = pltpu.async_copy(vals_hbm.at[idx_s.at[b0]], rows.at[0], sem0)
          d1 = pltpu.async_copy(vals_hbm.at[idx_s.at[b1]], rows.at[1], sem1)
          d0.wait()
          pltpu.sync_copy(rows.at[0], acc.at[idx_d.at[b0]], add=True)
          d1.wait()
          pltpu.sync_copy(rows.at[1], acc.at[idx_d.at[b1]], add=True)
          return 0

        lax.fori_loop(0, NT, body, 0)

      plsc.subcore_barrier()
      for q in range(RP // 128):
        pltpu.sync_copy(acc.at[seq_v.at[q]], rows.at[0])
        pltpu.sync_copy(rows.at[0],
                        out_hbm.at[pl.ds(base + s * RP + q * 128, 128)])
      plsc.subcore_barrier()

  return agg


_agg2 = _make_agg_kernel(2)
_agg4 = _make_agg_kernel(4)


def _dis_body(p0, p1, o):
  deg = p0[:, :1] + p1[:, :1] + 1.0
  o[...] = lax.rsqrt(jnp.maximum(deg, 1.0))


def _dis(p0, p1):
  mb = 1024
  return pl.pallas_call(
      _dis_body,
      grid=(R // mb,),
      in_specs=[
          pl.BlockSpec((mb, 128), lambda i: (i, 0)),
          pl.BlockSpec((mb, 128), lambda i: (i, 0)),
      ],
      out_specs=pl.BlockSpec((mb, 1), lambda i: (i, 0)),
      out_shape=jax.ShapeDtypeStruct((R, 1), jnp.float32),
  )(p0, p1)


def _prescale_body(x, d, o):
  o[0] = x[...] * d[...]


def _prescale(x, dis):
  mb = 400
  return pl.pallas_call(
      _prescale_body,
      grid=(N // mb, 2),
      in_specs=[
          pl.BlockSpec((mb, 128), lambda i, c: (i, c)),
          pl.BlockSpec((mb, 1), lambda i, c: (i, 0)),
      ],
      out_specs=pl.BlockSpec((1, mb, 128), lambda i, c: (c, i, 0)),
      out_shape=jax.ShapeDtypeStruct((2, R, 128), jnp.float32),
  )(x, dis)


def _mm1_body(a, w, d, b, o, acc):
  k = pl.program_id(2)

  @pl.when(k == 0)
  def _():
    acc[...] = jnp.zeros_like(acc)

  acc[...] += jnp.dot(a[0] * d[...], w[0],
                      preferred_element_type=jnp.float32)

  @pl.when(k == 1)
  def _():
    o[...] = jnp.maximum(acc[...] + b[...], 0.0)


def _mm1(agg1, w1r, dis, b1):
  mb, nb = 400, 256
  return pl.pallas_call(
      _mm1_body,
      grid=(N // mb, 1024 // nb, 2),
      in_specs=[
          pl.BlockSpec((1, mb, 128), lambda i, j, k: (k, i, 0)),
          pl.BlockSpec((1, 128, nb), lambda i, j, k: (k, 0, j)),
          pl.BlockSpec((mb, 1), lambda i, j, k: (i, 0)),
          pl.BlockSpec((1, nb), lambda i, j, k: (0, j)),
      ],
      out_specs=pl.BlockSpec((mb, nb), lambda i, j, k: (i, j)),
      out_shape=jax.ShapeDtypeStruct((N, 1024), jnp.float32),
      scratch_shapes=[pltpu.VMEM((mb, nb), jnp.float32)],
  )(agg1, w1r, dis, b1)


def _mm2_body(a, w, d, o):
  o[0] = jnp.dot(a[...], w[...],
                 preferred_element_type=jnp.float32) * d[...]


def _mm2(h, w2, dis):
  mb = 400
  return pl.pallas_call(
      _mm2_body,
      grid=(N // mb, 4),
      in_specs=[
          pl.BlockSpec((mb, 1024), lambda i, j: (i, 0)),
          pl.BlockSpec((1024, 128), lambda i, j: (0, j)),
          pl.BlockSpec((mb, 1), lambda i, j: (i, 0)),
      ],
      out_specs=pl.BlockSpec((1, mb, 128), lambda i, j: (j, i, 0)),
      out_shape=jax.ShapeDtypeStruct((4, R, 128), jnp.float32),
  )(h, w2, dis)


def _final_body(a, d, b, o):
  o[...] = a[0] * d[...] + b[0]


def _final(agg2, dis, b2r):
  mb = 400
  return pl.pallas_call(
      _final_body,
      grid=(N // mb, 4),
      in_specs=[
          pl.BlockSpec((1, mb, 128), lambda i, c: (c, i, 0)),
          pl.BlockSpec((mb, 1), lambda i, c: (i, 0)),
          pl.BlockSpec((1, 1, 128), lambda i, c: (c, 0, 0)),
      ],
      out_specs=pl.BlockSpec((mb, 128), lambda i, c: (i, c)),
      out_shape=jax.ShapeDtypeStruct((N, 512), jnp.float32),
  )(agg2, dis, b2r)


def kernel(x, edge_index, W1, b1, W2, b2):
  src = edge_index[0].astype(jnp.int32)
  dst = edge_index[1].astype(jnp.int32)

  src_t = jnp.pad(src.reshape(NS, EPT), ((0, 0), (0, NB * 128 - EPT)))
  src_t = src_t.reshape(NS, NB, 128)
  dst_t = jnp.pad(dst.reshape(NS, EPT), ((0, 0), (0, NB * 128 - EPT)),
                  constant_values=N).reshape(NS, NB, 128)
  off2 = (R * jnp.arange(2, dtype=jnp.int32))[:, None, None, None]
  off4 = (R * jnp.arange(4, dtype=jnp.int32))[:, None, None, None]
  src2 = src_t[None] + off2
  src4 = src_t[None] + off4

  dst_d = jnp.pad(dst.reshape(NC * NS, EPW),
                  ((0, 0), (0, NBD * 128 - EPW)),
                  constant_values=N).reshape(NC * NS, NBD, 128)

  seq = jnp.arange(R, dtype=jnp.int32).reshape(NS, RP // 128, 128)

  ones_in = jnp.ones((128, 128), jnp.float32)
  zeros_in = jnp.zeros((128, 128), jnp.float32)
  parts = _deg_kernel(dst_d, seq, ones_in, zeros_in).reshape(NC, R, 128)
  dis = _dis(parts[0], parts[1])

  vals1 = _prescale(x, dis).reshape(2 * R, 128)
  agg1 = _agg2(vals1, src2, dst_t, seq)
  h = _mm1(agg1.reshape(2, R, 128), W1.reshape(2, 128, 1024), dis,
           b1.reshape(1, 1024))
  vals2 = _mm2(h, W2, dis).reshape(4 * R, 128)
  agg2 = _agg4(vals2, src4, dst_t, seq)
  return _final(agg2.reshape(4, R, 128), dis, b2.reshape(4, 1, 128))


# speedup vs baseline: 6.5579x; 1.0135x over previous
import functools

import jax
import jax.numpy as jnp
from jax import lax
from jax.experimental import pallas as pl
from jax.experimental.pallas import tpu as pltpu
from jax.experimental.pallas import tpu_sc as plsc

N = 10000
E = 160000
NC = 2
NS = 16
R = 10240
RP = R // NS
EPT = E // NS
NB = 80
NBH = NB // 2
NT = NBH // 2
EPW = E // (NC * NS)
NBD = 40

_mesh = plsc.VectorSubcoreMesh(
    core_axis_name="c", subcore_axis_name="s", num_cores=NC, num_subcores=NS)


@functools.partial(
    pl.kernel,
    out_type=jax.ShapeDtypeStruct((NC * R, 128), jnp.float32),
    mesh=_mesh,
    scratch_types=[
        pltpu.VMEM((NBD, 128), jnp.int32),
        pltpu.VMEM((RP // 128, 128), jnp.int32),
        pltpu.VMEM((128, 128), jnp.float32),
        pltpu.VMEM((128, 128), jnp.float32),
        pltpu.VMEM_SHARED((R, 128), jnp.float32),
    ],
)
def _deg_kernel(dst_hbm, seq_hbm, ones_hbm, zeros_hbm, out_hbm,
                idx_v, seq_v, ones_v, stage_v, acc):
  c = lax.axis_index("c")
  s = lax.axis_index("s")
  w = s * NC + c

  pltpu.sync_copy(ones_hbm, ones_v)
  pltpu.sync_copy(zeros_hbm, stage_v)
  pltpu.sync_copy(dst_hbm.at[w], idx_v)
  pltpu.sync_copy(seq_hbm.at[s], seq_v)
  for q in range(RP // 128):
    pltpu.sync_copy(stage_v, acc.at[seq_v.at[q]])
  plsc.subcore_barrier()

  def body(b, _):
    pltpu.sync_copy(ones_v, acc.at[idx_v.at[b]], add=True)
    return 0
  lax.fori_loop(0, NBD, body, 0)

  plsc.subcore_barrier()
  for q in range(RP // 128):
    pltpu.sync_copy(acc.at[seq_v.at[q]], stage_v)
    pltpu.sync_copy(stage_v,
                    out_hbm.at[pl.ds(c * R + s * RP + q * 128, 128)])


def _make_agg_kernel(nch):
  cps = nch // NC

  @functools.partial(
      pl.kernel,
      out_type=jax.ShapeDtypeStruct((nch * R, 128), jnp.float32),
      mesh=_mesh,
      scratch_types=[
          pltpu.VMEM((NBH, 128), jnp.int32),
          pltpu.VMEM((NBH, 128), jnp.int32),
          pltpu.VMEM((RP // 128, 128), jnp.int32),
          pltpu.VMEM((2, 128, 128), jnp.float32),
          pltpu.VMEM_SHARED((R, 128), jnp.float32),
          pltpu.SemaphoreType.DMA,
          pltpu.SemaphoreType.DMA,
          pltpu.SemaphoreType.DMA,
          pltpu.SemaphoreType.DMA,
      ],
  )
  def agg(vals_hbm, src_hbm, dst_hbm, seq_hbm, out_hbm,
          idx_s, idx_d, seq_v, rows, acc, semg0, semg1, sems0, sems1):
    c = lax.axis_index("c")
    s = lax.axis_index("s")
    pltpu.sync_copy(seq_hbm.at[s], seq_v)

    for j in range(cps):
      chunk = c * cps + j
      base = chunk * R
      for q in range(RP // 128):
        pltpu.sync_copy(vals_hbm.at[pl.ds(base + s * RP + q * 128, 128)],
                        rows.at[0])
        pltpu.sync_copy(rows.at[0], acc.at[seq_v.at[q]])
      plsc.subcore_barrier()

      for half in range(2):
        pltpu.sync_copy(src_hbm.at[chunk, s, pl.ds(half * NBH, NBH)], idx_s)
        pltpu.sync_copy(dst_hbm.at[s, pl.ds(half * NBH, NBH)], idx_d)

        pltpu.async_copy(vals_hbm.at[idx_s.at[0]], rows.at[0], semg0)
        pltpu.async_copy(vals_hbm.at[idx_s.at[1]], rows.at[1], semg1)

        def body(t, _):
          b0 = 2 * t
          b1 = b0 + 1
          pltpu.make_async_copy(
              vals_hbm.at[idx_s.at[b0]], rows.at[0], semg0).wait()
          s0 = pltpu.async_copy(rows.at[0], acc.at[idx_d.at[b0]], sems0,
                                add=True)
          pltpu.make_async_copy(
              vals_hbm.at[idx_s.at[b1]], rows.at[1], semg1).wait()
          s1 = pltpu.async_copy(rows.at[1], acc.at[idx_d.at[b1]], sems1,
                                add=True)
          s0.wait()

          @pl.when(t + 1 < NT)
          def _():
            pltpu.async_copy(vals_hbm.at[idx_s.at[b0 + 2]], rows.at[0],
                             semg0)
          s1.wait()

          @pl.when(t + 1 < NT)
          def _():
            pltpu.async_copy(vals_hbm.at[idx_s.at[b1 + 2]], rows.at[1],
                             semg1)
          return 0

        lax.fori_loop(0, NT, body, 0)

      plsc.subcore_barrier()
      for q in range(RP // 128):
        pltpu.sync_copy(acc.at[seq_v.at[q]], rows.at[0])
        pltpu.sync_copy(rows.at[0],
                        out_hbm.at[pl.ds(base + s * RP + q * 128, 128)])
      plsc.subcore_barrier()

  return agg


_agg2 = _make_agg_kernel(2)
_agg4 = _make_agg_kernel(4)


def _dis_body(p0, p1, o):
  deg = p0[:, :1] + p1[:, :1] + 1.0
  o[...] = lax.rsqrt(jnp.maximum(deg, 1.0))


def _dis(p0, p1):
  mb = 1024
  return pl.pallas_call(
      _dis_body,
      grid=(R // mb,),
      in_specs=[
          pl.BlockSpec((mb, 128), lambda i: (i, 0)),
          pl.BlockSpec((mb, 128), lambda i: (i, 0)),
      ],
      out_specs=pl.BlockSpec((mb, 1), lambda i: (i, 0)),
      out_shape=jax.ShapeDtypeStruct((R, 1), jnp.float32),
  )(p0, p1)


def _prescale_body(x, d, o):
  o[0] = x[...] * d[...]


def _prescale(x, dis):
  mb = 400
  return pl.pallas_call(
      _prescale_body,
      grid=(N // mb, 2),
      in_specs=[
          pl.BlockSpec((mb, 128), lambda i, c: (i, c)),
          pl.BlockSpec((mb, 1), lambda i, c: (i, 0)),
      ],
      out_specs=pl.BlockSpec((1, mb, 128), lambda i, c: (c, i, 0)),
      out_shape=jax.ShapeDtypeStruct((2, R, 128), jnp.float32),
  )(x, dis)


def _mm1_body(a, w, d, b, o, acc):
  k = pl.program_id(2)

  @pl.when(k == 0)
  def _():
    acc[...] = jnp.zeros_like(acc)

  acc[...] += jnp.dot(a[0] * d[...], w[0],
                      preferred_element_type=jnp.float32)

  @pl.when(k == 1)
  def _():
    o[...] = jnp.maximum(acc[...] + b[...], 0.0)


def _mm1(agg1, w1r, dis, b1):
  mb, nb = 400, 256
  return pl.pallas_call(
      _mm1_body,
      grid=(N // mb, 1024 // nb, 2),
      in_specs=[
          pl.BlockSpec((1, mb, 128), lambda i, j, k: (k, i, 0)),
          pl.BlockSpec((1, 128, nb), lambda i, j, k: (k, 0, j)),
          pl.BlockSpec((mb, 1), lambda i, j, k: (i, 0)),
          pl.BlockSpec((1, nb), lambda i, j, k: (0, j)),
      ],
      out_specs=pl.BlockSpec((mb, nb), lambda i, j, k: (i, j)),
      out_shape=jax.ShapeDtypeStruct((N, 1024), jnp.float32),
      scratch_shapes=[pltpu.VMEM((mb, nb), jnp.float32)],
  )(agg1, w1r, dis, b1)


def _mm2_body(a, w, d, o):
  o[0] = jnp.dot(a[...], w[...],
                 preferred_element_type=jnp.float32) * d[...]


def _mm2(h, w2, dis):
  mb = 400
  return pl.pallas_call(
      _mm2_body,
      grid=(N // mb, 4),
      in_specs=[
          pl.BlockSpec((mb, 1024), lambda i, j: (i, 0)),
          pl.BlockSpec((1024, 128), lambda i, j: (0, j)),
          pl.BlockSpec((mb, 1), lambda i, j: (i, 0)),
      ],
      out_specs=pl.BlockSpec((1, mb, 128), lambda i, j: (j, i, 0)),
      out_shape=jax.ShapeDtypeStruct((4, R, 128), jnp.float32),
  )(h, w2, dis)


def _final_body(a, d, b, o):
  o[...] = a[0] * d[...] + b[0]


def _final(agg2, dis, b2r):
  mb = 400
  return pl.pallas_call(
      _final_body,
      grid=(N // mb, 4),
      in_specs=[
          pl.BlockSpec((1, mb, 128), lambda i, c: (c, i, 0)),
          pl.BlockSpec((mb, 1), lambda i, c: (i, 0)),
          pl.BlockSpec((1, 1, 128), lambda i, c: (c, 0, 0)),
      ],
      out_specs=pl.BlockSpec((mb, 128), lambda i, c: (i, c)),
      out_shape=jax.ShapeDtypeStruct((N, 512), jnp.float32),
  )(agg2, dis, b2r)


def kernel(x, edge_index, W1, b1, W2, b2):
  src = edge_index[0].astype(jnp.int32)
  dst = edge_index[1].astype(jnp.int32)

  src_t = jnp.pad(src.reshape(NS, EPT), ((0, 0), (0, NB * 128 - EPT)))
  src_t = src_t.reshape(NS, NB, 128)
  dst_t = jnp.pad(dst.reshape(NS, EPT), ((0, 0), (0, NB * 128 - EPT)),
                  constant_values=N).reshape(NS, NB, 128)
  off2 = (R * jnp.arange(2, dtype=jnp.int32))[:, None, None, None]
  off4 = (R * jnp.arange(4, dtype=jnp.int32))[:, None, None, None]
  src2 = src_t[None] + off2
  src4 = src_t[None] + off4

  dst_d = jnp.pad(dst.reshape(NC * NS, EPW),
                  ((0, 0), (0, NBD * 128 - EPW)),
                  constant_values=N).reshape(NC * NS, NBD, 128)

  seq = jnp.arange(R, dtype=jnp.int32).reshape(NS, RP // 128, 128)

  ones_in = jnp.ones((128, 128), jnp.float32)
  zeros_in = jnp.zeros((128, 128), jnp.float32)
  parts = _deg_kernel(dst_d, seq, ones_in, zeros_in).reshape(NC, R, 128)
  dis = _dis(parts[0], parts[1])

  vals1 = _prescale(x, dis).reshape(2 * R, 128)
  agg1 = _agg2(vals1, src2, dst_t, seq)
  h = _mm1(agg1.reshape(2, R, 128), W1.reshape(2, 128, 1024), dis,
           b1.reshape(1, 1024))
  vals2 = _mm2(h, W2, dis).reshape(4 * R, 128)
  agg2 = _agg4(vals2, src4, dst_t, seq)
  return _final(agg2.reshape(4, R, 128), dis, b2.reshape(4, 1, 128))


# speedup vs baseline: 6.8605x; 1.0461x over previous
import functools

import jax
import jax.numpy as jnp
from jax import lax
from jax.experimental import pallas as pl
from jax.experimental.pallas import tpu as pltpu
from jax.experimental.pallas import tpu_sc as plsc

N = 10000
E = 160000
NC = 2
NS = 16
R = 10240
RP = R // NS
EPT = E // NS
NB = 80
NBH = NB // 2
NT = NBH // 2
EPW = E // (NC * NS)
NBD = 40

_mesh = plsc.VectorSubcoreMesh(
    core_axis_name="c", subcore_axis_name="s", num_cores=NC, num_subcores=NS)


@functools.partial(
    pl.kernel,
    out_type=jax.ShapeDtypeStruct((NC * R, 128), jnp.float32),
    mesh=_mesh,
    scratch_types=[
        pltpu.VMEM((NBD, 128), jnp.int32),
        pltpu.VMEM((RP // 128, 128), jnp.int32),
        pltpu.VMEM((128, 128), jnp.float32),
        pltpu.VMEM((128, 128), jnp.float32),
        pltpu.VMEM_SHARED((R, 128), jnp.float32),
    ],
)
def _deg_kernel(dst_hbm, seq_hbm, ones_hbm, zeros_hbm, out_hbm,
                idx_v, seq_v, ones_v, stage_v, acc):
  c = lax.axis_index("c")
  s = lax.axis_index("s")
  w = s * NC + c

  pltpu.sync_copy(ones_hbm, ones_v)
  pltpu.sync_copy(zeros_hbm, stage_v)
  pltpu.sync_copy(dst_hbm.at[w], idx_v)
  pltpu.sync_copy(seq_hbm.at[s], seq_v)
  for q in range(RP // 128):
    pltpu.sync_copy(stage_v, acc.at[seq_v.at[q]])
  plsc.subcore_barrier()

  def body(b, _):
    pltpu.sync_copy(ones_v, acc.at[idx_v.at[b]], add=True)
    return 0
  lax.fori_loop(0, NBD, body, 0)

  plsc.subcore_barrier()
  for q in range(RP // 128):
    pltpu.sync_copy(acc.at[seq_v.at[q]], stage_v)
    pltpu.sync_copy(stage_v,
                    out_hbm.at[pl.ds(c * R + s * RP + q * 128, 128)])


def _make_agg_kernel(nch):
  cps = nch // NC

  @functools.partial(
      pl.kernel,
      out_type=jax.ShapeDtypeStruct((nch * R, 128), jnp.float32),
      mesh=_mesh,
      scratch_types=[
          pltpu.VMEM((NBH, 128), jnp.int32),
          pltpu.VMEM((NBH, 128), jnp.int32),
          pltpu.VMEM((RP // 128, 128), jnp.int32),
          pltpu.VMEM((2, 128, 128), jnp.float32),
          pltpu.VMEM_SHARED((R, 128), jnp.float32),
          pltpu.SemaphoreType.DMA,
          pltpu.SemaphoreType.DMA,
          pltpu.SemaphoreType.DMA,
          pltpu.SemaphoreType.DMA,
      ],
  )
  def agg(vals_hbm, src_hbm, dst_hbm, seq_hbm, out_hbm,
          idx_s, idx_d, seq_v, rows, acc, semg0, semg1, sems0, sems1):
    c = lax.axis_index("c")
    s = lax.axis_index("s")
    pltpu.sync_copy(seq_hbm.at[s], seq_v)

    for j in range(cps):
      chunk = c * cps + j
      base = chunk * R
      for q in range(RP // 128):
        pltpu.sync_copy(vals_hbm.at[pl.ds(base + s * RP + q * 128, 128)],
                        rows.at[0])
        pltpu.sync_copy(rows.at[0], acc.at[seq_v.at[q]])
      plsc.subcore_barrier()

      for half in range(2):
        pltpu.sync_copy(src_hbm.at[chunk, s, pl.ds(half * NBH, NBH)], idx_s)
        pltpu.sync_copy(dst_hbm.at[s, pl.ds(half * NBH, NBH)], idx_d)

        pltpu.async_copy(vals_hbm.at[idx_s.at[0]], rows.at[0], semg0)
        pltpu.async_copy(vals_hbm.at[idx_s.at[1]], rows.at[1], semg1)

        def body(t, _):
          b0 = 2 * t
          b1 = b0 + 1
          pltpu.make_async_copy(
              vals_hbm.at[idx_s.at[b0]], rows.at[0], semg0).wait()
          s0 = pltpu.async_copy(rows.at[0], acc.at[idx_d.at[b0]], sems0,
                                add=True)
          pltpu.make_async_copy(
              vals_hbm.at[idx_s.at[b1]], rows.at[1], semg1).wait()
          s1 = pltpu.async_copy(rows.at[1], acc.at[idx_d.at[b1]], sems1,
                                add=True)
          s0.wait()

          @pl.when(t + 1 < NT)
          def _():
            pltpu.async_copy(vals_hbm.at[idx_s.at[b0 + 2]], rows.at[0],
                             semg0)
          s1.wait()

          @pl.when(t + 1 < NT)
          def _():
            pltpu.async_copy(vals_hbm.at[idx_s.at[b1 + 2]], rows.at[1],
                             semg1)
          return 0

        lax.fori_loop(0, NT, body, 0)

      plsc.subcore_barrier()
      for q in range(RP // 128):
        pltpu.sync_copy(acc.at[seq_v.at[q]], rows.at[0])
        pltpu.sync_copy(rows.at[0],
                        out_hbm.at[pl.ds(base + s * RP + q * 128, 128)])
      plsc.subcore_barrier()

  return agg


_agg2 = _make_agg_kernel(2)
_agg4 = _make_agg_kernel(4)


def _dis_body(p0, p1, o):
  deg = p0[:, :1] + p1[:, :1] + 1.0
  o[...] = lax.rsqrt(jnp.maximum(deg, 1.0))


def _dis(p0, p1):
  mb = 1024
  return pl.pallas_call(
      _dis_body,
      grid=(R // mb,),
      in_specs=[
          pl.BlockSpec((mb, 128), lambda i: (i, 0)),
          pl.BlockSpec((mb, 128), lambda i: (i, 0)),
      ],
      out_specs=pl.BlockSpec((mb, 1), lambda i: (i, 0)),
      out_shape=jax.ShapeDtypeStruct((R, 1), jnp.float32),
  )(p0, p1)


def _prescale_body(x, d, o):
  o[0] = x[...] * d[...]


def _prescale(x, dis):
  mb = 400
  return pl.pallas_call(
      _prescale_body,
      grid=(N // mb, 2),
      in_specs=[
          pl.BlockSpec((mb, 128), lambda i, c: (i, c)),
          pl.BlockSpec((mb, 1), lambda i, c: (i, 0)),
      ],
      out_specs=pl.BlockSpec((1, mb, 128), lambda i, c: (c, i, 0)),
      out_shape=jax.ShapeDtypeStruct((2, R, 128), jnp.float32),
  )(x, dis)


def _mm1_body(a0, a1, w0, w1, d, b, o):
  o[...] = jnp.maximum(
      jnp.dot(a0[...] * d[...], w0[...], preferred_element_type=jnp.float32)
      + jnp.dot(a1[...] * d[...], w1[...],
                preferred_element_type=jnp.float32)
      + b[...], 0.0)


def _mm1(agg1_0, agg1_1, w1_0, w1_1, dis, b1):
  mb, nb = 400, 256
  return pl.pallas_call(
      _mm1_body,
      grid=(N // mb, 1024 // nb),
      in_specs=[
          pl.BlockSpec((mb, 128), lambda i, j: (i, 0)),
          pl.BlockSpec((mb, 128), lambda i, j: (i, 0)),
          pl.BlockSpec((128, nb), lambda i, j: (0, j)),
          pl.BlockSpec((128, nb), lambda i, j: (0, j)),
          pl.BlockSpec((mb, 1), lambda i, j: (i, 0)),
          pl.BlockSpec((1, nb), lambda i, j: (0, j)),
      ],
      out_specs=pl.BlockSpec((mb, nb), lambda i, j: (i, j)),
      out_shape=jax.ShapeDtypeStruct((N, 1024), jnp.float32),
  )(agg1_0, agg1_1, w1_0, w1_1, dis, b1)


def _mm2_body(a, w, d, o):
  o[0] = jnp.dot(a[...], w[...],
                 preferred_element_type=jnp.float32) * d[...]


def _mm2(h, w2, dis):
  mb = 400
  return pl.pallas_call(
      _mm2_body,
      grid=(N // mb, 4),
      in_specs=[
          pl.BlockSpec((mb, 1024), lambda i, j: (i, 0)),
          pl.BlockSpec((1024, 128), lambda i, j: (0, j)),
          pl.BlockSpec((mb, 1), lambda i, j: (i, 0)),
      ],
      out_specs=pl.BlockSpec((1, mb, 128), lambda i, j: (j, i, 0)),
      out_shape=jax.ShapeDtypeStruct((4, R, 128), jnp.float32),
  )(h, w2, dis)


def _final_body(a, d, b, o):
  o[...] = a[0] * d[...] + b[0]


def _final(agg2, dis, b2r):
  mb = 400
  return pl.pallas_call(
      _final_body,
      grid=(N // mb, 4),
      in_specs=[
          pl.BlockSpec((1, mb, 128), lambda i, c: (c, i, 0)),
          pl.BlockSpec((mb, 1), lambda i, c: (i, 0)),
          pl.BlockSpec((1, 1, 128), lambda i, c: (c, 0, 0)),
      ],
      out_specs=pl.BlockSpec((mb, 128), lambda i, c: (i, c)),
      out_shape=jax.ShapeDtypeStruct((N, 512), jnp.float32),
  )(agg2, dis, b2r)


def kernel(x, edge_index, W1, b1, W2, b2):
  src = edge_index[0].astype(jnp.int32)
  dst = edge_index[1].astype(jnp.int32)

  src_t = jnp.pad(src.reshape(NS, EPT), ((0, 0), (0, NB * 128 - EPT)))
  src_t = src_t.reshape(NS, NB, 128)
  dst_t = jnp.pad(dst.reshape(NS, EPT), ((0, 0), (0, NB * 128 - EPT)),
                  constant_values=N).reshape(NS, NB, 128)
  off2 = (R * jnp.arange(2, dtype=jnp.int32))[:, None, None, None]
  off4 = (R * jnp.arange(4, dtype=jnp.int32))[:, None, None, None]
  src2 = src_t[None] + off2
  src4 = src_t[None] + off4

  dst_d = jnp.pad(dst.reshape(NC * NS, EPW),
                  ((0, 0), (0, NBD * 128 - EPW)),
                  constant_values=N).reshape(NC * NS, NBD, 128)

  seq = jnp.arange(R, dtype=jnp.int32).reshape(NS, RP // 128, 128)

  ones_in = jnp.ones((128, 128), jnp.float32)
  zeros_in = jnp.zeros((128, 128), jnp.float32)
  parts = _deg_kernel(dst_d, seq, ones_in, zeros_in).reshape(NC, R, 128)
  dis = _dis(parts[0], parts[1])

  vals1 = _prescale(x, dis).reshape(2 * R, 128)
  agg1 = _agg2(vals1, src2, dst_t, seq)
  a1r = agg1.reshape(2, R, 128)
  h = _mm1(a1r[0], a1r[1], W1[:128], W1[128:], dis,
           b1.reshape(1, 1024))
  vals2 = _mm2(h, W2, dis).reshape(4 * R, 128)
  agg2 = _agg4(vals2, src4, dst_t, seq)
  return _final(agg2.reshape(4, R, 128), dis, b2.reshape(4, 1, 128))


# speedup vs baseline: 7.0767x; 1.0315x over previous
import functools

import jax
import jax.numpy as jnp
from jax import lax
from jax.experimental import pallas as pl
from jax.experimental.pallas import tpu as pltpu
from jax.experimental.pallas import tpu_sc as plsc

N = 10000
E = 160000
NC = 2
NS = 16
R = 10240
RP = R // NS
EPT = E // NS
NB = 80
NBH = NB // 2
NT = NBH // 2
EPW = E // (NC * NS)
NBD = 40

_mesh = plsc.VectorSubcoreMesh(
    core_axis_name="c", subcore_axis_name="s", num_cores=NC, num_subcores=NS)


@functools.partial(
    pl.kernel,
    out_type=jax.ShapeDtypeStruct((NC * R, 128), jnp.float32),
    mesh=_mesh,
    scratch_types=[
        pltpu.VMEM((NBD, 128), jnp.int32),
        pltpu.VMEM((RP // 128, 128), jnp.int32),
        pltpu.VMEM((128, 128), jnp.float32),
        pltpu.VMEM((128, 128), jnp.float32),
        pltpu.VMEM_SHARED((R, 128), jnp.float32),
    ],
)
def _deg_kernel(dst_hbm, seq_hbm, ones_hbm, zeros_hbm, out_hbm,
                idx_v, seq_v, ones_v, stage_v, acc):
  c = lax.axis_index("c")
  s = lax.axis_index("s")
  w = s * NC + c

  pltpu.sync_copy(ones_hbm, ones_v)
  pltpu.sync_copy(zeros_hbm, stage_v)
  pltpu.sync_copy(dst_hbm.at[w], idx_v)
  pltpu.sync_copy(seq_hbm.at[s], seq_v)
  for q in range(RP // 128):
    pltpu.sync_copy(stage_v, acc.at[seq_v.at[q]])
  plsc.subcore_barrier()

  def body(b, _):
    pltpu.sync_copy(ones_v, acc.at[idx_v.at[b]], add=True)
    return 0
  lax.fori_loop(0, NBD, body, 0)

  plsc.subcore_barrier()
  for q in range(RP // 128):
    pltpu.sync_copy(acc.at[seq_v.at[q]], stage_v)
    pltpu.sync_copy(stage_v,
                    out_hbm.at[pl.ds(c * R + s * RP + q * 128, 128)])


def _make_agg_kernel(nch):
  cps = nch // NC

  @functools.partial(
      pl.kernel,
      out_type=jax.ShapeDtypeStruct((nch * R, 128), jnp.float32),
      mesh=_mesh,
      scratch_types=[
          pltpu.VMEM((NBH, 128), jnp.int32),
          pltpu.VMEM((NBH, 128), jnp.int32),
          pltpu.VMEM((RP // 128, 128), jnp.int32),
          pltpu.VMEM((2, 128, 128), jnp.float32),
          pltpu.VMEM_SHARED((R, 128), jnp.float32),
          pltpu.SemaphoreType.DMA,
          pltpu.SemaphoreType.DMA,
          pltpu.SemaphoreType.DMA,
          pltpu.SemaphoreType.DMA,
      ],
  )
  def agg(vals_hbm, src_hbm, dst_hbm, seq_hbm, out_hbm,
          idx_s, idx_d, seq_v, rows, acc, semg0, semg1, sems0, sems1):
    c = lax.axis_index("c")
    s = lax.axis_index("s")
    pltpu.sync_copy(seq_hbm.at[s], seq_v)

    for j in range(cps):
      chunk = c * cps + j
      base = chunk * R
      for q in range(RP // 128):
        pltpu.sync_copy(vals_hbm.at[pl.ds(base + s * RP + q * 128, 128)],
                        rows.at[0])
        pltpu.sync_copy(rows.at[0], acc.at[seq_v.at[q]])
      plsc.subcore_barrier()

      for half in range(2):
        pltpu.sync_copy(src_hbm.at[chunk, s, pl.ds(half * NBH, NBH)], idx_s)
        pltpu.sync_copy(dst_hbm.at[s, pl.ds(half * NBH, NBH)], idx_d)

        pltpu.async_copy(vals_hbm.at[idx_s.at[0]], rows.at[0], semg0)
        pltpu.async_copy(vals_hbm.at[idx_s.at[1]], rows.at[1], semg1)

        def body(t, _):
          b0 = 2 * t
          b1 = b0 + 1
          pltpu.make_async_copy(
              vals_hbm.at[idx_s.at[b0]], rows.at[0], semg0).wait()
          s0 = pltpu.async_copy(rows.at[0], acc.at[idx_d.at[b0]], sems0,
                                add=True)
          pltpu.make_async_copy(
              vals_hbm.at[idx_s.at[b1]], rows.at[1], semg1).wait()
          s1 = pltpu.async_copy(rows.at[1], acc.at[idx_d.at[b1]], sems1,
                                add=True)
          s0.wait()

          @pl.when(t + 1 < NT)
          def _():
            pltpu.async_copy(vals_hbm.at[idx_s.at[b0 + 2]], rows.at[0],
                             semg0)
          s1.wait()

          @pl.when(t + 1 < NT)
          def _():
            pltpu.async_copy(vals_hbm.at[idx_s.at[b1 + 2]], rows.at[1],
                             semg1)
          return 0

        lax.fori_loop(0, NT, body, 0)

      plsc.subcore_barrier()
      for q in range(RP // 128):
        pltpu.sync_copy(acc.at[seq_v.at[q]], rows.at[0])
        pltpu.sync_copy(rows.at[0],
                        out_hbm.at[pl.ds(base + s * RP + q * 128, 128)])
      plsc.subcore_barrier()

  return agg


_agg2 = _make_agg_kernel(2)
_agg4 = _make_agg_kernel(4)


def _predis_body(p0, p1, x, od, ov):
  deg = p0[:, :1] + p1[:, :1] + 1.0
  d = lax.rsqrt(deg)
  od[...] = d
  ov[0] = x[...] * d


def _predis(p0, p1, x):
  mb = 400
  return pl.pallas_call(
      _predis_body,
      grid=(N // mb, 2),
      in_specs=[
          pl.BlockSpec((mb, 128), lambda i, c: (i, 0)),
          pl.BlockSpec((mb, 128), lambda i, c: (i, 0)),
          pl.BlockSpec((mb, 128), lambda i, c: (i, c)),
      ],
      out_specs=[
          pl.BlockSpec((mb, 1), lambda i, c: (i, 0)),
          pl.BlockSpec((1, mb, 128), lambda i, c: (c, i, 0)),
      ],
      out_shape=[
          jax.ShapeDtypeStruct((R, 1), jnp.float32),
          jax.ShapeDtypeStruct((2, R, 128), jnp.float32),
      ],
  )(p0, p1, x)


def _mm1_body(a0, a1, w0, w1, d, b, o):
  o[...] = jnp.maximum(
      jnp.dot(a0[...] * d[...], w0[...], preferred_element_type=jnp.float32)
      + jnp.dot(a1[...] * d[...], w1[...],
                preferred_element_type=jnp.float32)
      + b[...], 0.0)


def _mm1(agg1_0, agg1_1, w1_0, w1_1, dis, b1):
  mb, nb = 400, 256
  return pl.pallas_call(
      _mm1_body,
      grid=(N // mb, 1024 // nb),
      in_specs=[
          pl.BlockSpec((mb, 128), lambda i, j: (i, 0)),
          pl.BlockSpec((mb, 128), lambda i, j: (i, 0)),
          pl.BlockSpec((128, nb), lambda i, j: (0, j)),
          pl.BlockSpec((128, nb), lambda i, j: (0, j)),
          pl.BlockSpec((mb, 1), lambda i, j: (i, 0)),
          pl.BlockSpec((1, nb), lambda i, j: (0, j)),
      ],
      out_specs=pl.BlockSpec((mb, nb), lambda i, j: (i, j)),
      out_shape=jax.ShapeDtypeStruct((N, 1024), jnp.float32),
  )(agg1_0, agg1_1, w1_0, w1_1, dis, b1)


def _mm2_body(a, w, d, o):
  r = jnp.dot(a[...], w[...], preferred_element_type=jnp.float32) * d[...]
  o[0] = r[:, :128]
  o[1] = r[:, 128:]


def _mm2(h, w2, dis):
  mb = 400
  return pl.pallas_call(
      _mm2_body,
      grid=(N // mb, 2),
      in_specs=[
          pl.BlockSpec((mb, 1024), lambda i, j: (i, 0)),
          pl.BlockSpec((1024, 256), lambda i, j: (0, j)),
          pl.BlockSpec((mb, 1), lambda i, j: (i, 0)),
      ],
      out_specs=pl.BlockSpec((2, mb, 128), lambda i, j: (j, i, 0)),
      out_shape=jax.ShapeDtypeStruct((4, R, 128), jnp.float32),
  )(h, w2, dis)


def _final_body(a, d, b, o):
  o[...] = a[0] * d[...] + b[0]


def _final(agg2, dis, b2r):
  mb = 400
  return pl.pallas_call(
      _final_body,
      grid=(N // mb, 4),
      in_specs=[
          pl.BlockSpec((1, mb, 128), lambda i, c: (c, i, 0)),
          pl.BlockSpec((mb, 1), lambda i, c: (i, 0)),
          pl.BlockSpec((1, 1, 128), lambda i, c: (c, 0, 0)),
      ],
      out_specs=pl.BlockSpec((mb, 128), lambda i, c: (i, c)),
      out_shape=jax.ShapeDtypeStruct((N, 512), jnp.float32),
  )(agg2, dis, b2r)


def kernel(x, edge_index, W1, b1, W2, b2):
  src = edge_index[0].astype(jnp.int32)
  dst = edge_index[1].astype(jnp.int32)

  src_t = jnp.pad(src.reshape(NS, EPT), ((0, 0), (0, NB * 128 - EPT)))
  src_t = src_t.reshape(NS, NB, 128)
  dst_t = jnp.pad(dst.reshape(NS, EPT), ((0, 0), (0, NB * 128 - EPT)),
                  constant_values=N).reshape(NS, NB, 128)
  off2 = (R * jnp.arange(2, dtype=jnp.int32))[:, None, None, None]
  off4 = (R * jnp.arange(4, dtype=jnp.int32))[:, None, None, None]
  src2 = src_t[None] + off2
  src4 = src_t[None] + off4

  dst_d = jnp.pad(dst.reshape(NC * NS, EPW),
                  ((0, 0), (0, NBD * 128 - EPW)),
                  constant_values=N).reshape(NC * NS, NBD, 128)

  seq = jnp.arange(R, dtype=jnp.int32).reshape(NS, RP // 128, 128)

  ones_in = jnp.ones((128, 128), jnp.float32)
  zeros_in = jnp.zeros((128, 128), jnp.float32)
  parts = _deg_kernel(dst_d, seq, ones_in, zeros_in).reshape(NC, R, 128)
  dis, vals1 = _predis(parts[0], parts[1], x)
  vals1 = vals1.reshape(2 * R, 128)
  agg1 = _agg2(vals1, src2, dst_t, seq)
  a1r = agg1.reshape(2, R, 128)
  h = _mm1(a1r[0], a1r[1], W1[:128], W1[128:], dis,
           b1.reshape(1, 1024))
  vals2 = _mm2(h, W2, dis).reshape(4 * R, 128)
  agg2 = _agg4(vals2, src4, dst_t, seq)
  return _final(agg2.reshape(4, R, 128), dis, b2.reshape(4, 1, 128))


# speedup vs baseline: 7.2216x; 1.0205x over previous
import functools

import jax
import jax.numpy as jnp
from jax import lax
from jax.experimental import pallas as pl
from jax.experimental.pallas import tpu as pltpu
from jax.experimental.pallas import tpu_sc as plsc

N = 10000
E = 160000
NC = 2
NS = 16
R = 10240
RP = R // NS
EPT = E // NS
NB = 80
NBH = NB // 2
NT = NBH // 2
EPW = E // (NC * NS)
NBD = 40

_mesh = plsc.VectorSubcoreMesh(
    core_axis_name="c", subcore_axis_name="s", num_cores=NC, num_subcores=NS)


@functools.partial(
    pl.kernel,
    out_type=jax.ShapeDtypeStruct((NC * R, 128), jnp.float32),
    mesh=_mesh,
    scratch_types=[
        pltpu.VMEM((NBD, 128), jnp.int32),
        pltpu.VMEM((RP // 128, 128), jnp.int32),
        pltpu.VMEM((128, 128), jnp.float32),
        pltpu.VMEM((128, 128), jnp.float32),
        pltpu.VMEM_SHARED((R, 128), jnp.float32),
    ],
)
def _deg_kernel(dst_hbm, seq_hbm, ones_hbm, zeros_hbm, out_hbm,
                idx_v, seq_v, ones_v, stage_v, acc):
  c = lax.axis_index("c")
  s = lax.axis_index("s")
  w = s * NC + c

  pltpu.sync_copy(ones_hbm, ones_v)
  pltpu.sync_copy(zeros_hbm, stage_v)
  pltpu.sync_copy(dst_hbm.at[w], idx_v)
  pltpu.sync_copy(seq_hbm.at[s], seq_v)
  for q in range(RP // 128):
    pltpu.sync_copy(stage_v, acc.at[seq_v.at[q]])
  plsc.subcore_barrier()

  def body(b, _):
    pltpu.sync_copy(ones_v, acc.at[idx_v.at[b]], add=True)
    return 0
  lax.fori_loop(0, NBD, body, 0)

  plsc.subcore_barrier()
  for q in range(RP // 128):
    pltpu.sync_copy(acc.at[seq_v.at[q]], stage_v)
    pltpu.sync_copy(stage_v,
                    out_hbm.at[pl.ds(c * R + s * RP + q * 128, 128)])


def _make_agg_kernel(nch):
  cps = nch // NC

  @functools.partial(
      pl.kernel,
      out_type=jax.ShapeDtypeStruct((nch * R, 128), jnp.float32),
      mesh=_mesh,
      scratch_types=[
          pltpu.VMEM((NBH, 128), jnp.int32),
          pltpu.VMEM((NBH, 128), jnp.int32),
          pltpu.VMEM((RP // 128, 128), jnp.int32),
          pltpu.VMEM((2, 128, 128), jnp.float32),
          pltpu.VMEM_SHARED((R, 128), jnp.float32),
          pltpu.SemaphoreType.DMA,
          pltpu.SemaphoreType.DMA,
          pltpu.SemaphoreType.DMA,
          pltpu.SemaphoreType.DMA,
      ],
  )
  def agg(vals_hbm, src_hbm, dst_hbm, seq_hbm, out_hbm,
          idx_s, idx_d, seq_v, rows, acc, semg0, semg1, sems0, sems1):
    c = lax.axis_index("c")
    s = lax.axis_index("s")
    pltpu.sync_copy(seq_hbm.at[s], seq_v)

    for j in range(cps):
      chunk = c * cps + j
      base = chunk * R
      for q in range(RP // 128):
        pltpu.sync_copy(vals_hbm.at[pl.ds(base + s * RP + q * 128, 128)],
                        rows.at[0])
        pltpu.sync_copy(rows.at[0], acc.at[seq_v.at[q]])
      plsc.subcore_barrier()

      for half in range(2):
        pltpu.sync_copy(src_hbm.at[chunk, s, pl.ds(half * NBH, NBH)], idx_s)
        pltpu.sync_copy(dst_hbm.at[s, pl.ds(half * NBH, NBH)], idx_d)

        pltpu.async_copy(vals_hbm.at[idx_s.at[0]], rows.at[0], semg0)
        pltpu.async_copy(vals_hbm.at[idx_s.at[1]], rows.at[1], semg1)

        def body(t, _):
          b0 = 2 * t
          b1 = b0 + 1
          pltpu.make_async_copy(
              vals_hbm.at[idx_s.at[b0]], rows.at[0], semg0).wait()
          s0 = pltpu.async_copy(rows.at[0], acc.at[idx_d.at[b0]], sems0,
                                add=True)
          pltpu.make_async_copy(
              vals_hbm.at[idx_s.at[b1]], rows.at[1], semg1).wait()
          s1 = pltpu.async_copy(rows.at[1], acc.at[idx_d.at[b1]], sems1,
                                add=True)
          s0.wait()

          @pl.when(t + 1 < NT)
          def _():
            pltpu.async_copy(vals_hbm.at[idx_s.at[b0 + 2]], rows.at[0],
                             semg0)
          s1.wait()

          @pl.when(t + 1 < NT)
          def _():
            pltpu.async_copy(vals_hbm.at[idx_s.at[b1 + 2]], rows.at[1],
                             semg1)
          return 0

        lax.fori_loop(0, NT, body, 0)

      plsc.subcore_barrier()
      for q in range(RP // 128):
        pltpu.sync_copy(acc.at[seq_v.at[q]], rows.at[0])
        pltpu.sync_copy(rows.at[0],
                        out_hbm.at[pl.ds(base + s * RP + q * 128, 128)])
      plsc.subcore_barrier()

  return agg


_agg2 = _make_agg_kernel(2)
_agg4 = _make_agg_kernel(4)


def _predis_body(p0, p1, x, od, ov):
  deg = p0[:, :1] + p1[:, :1] + 1.0
  d = lax.rsqrt(deg)
  od[...] = d
  ov[0] = x[...] * d


def _predis(p0, p1, x):
  mb = 400
  return pl.pallas_call(
      _predis_body,
      grid=(N // mb, 2),
      in_specs=[
          pl.BlockSpec((mb, 128), lambda i, c: (i, 0)),
          pl.BlockSpec((mb, 128), lambda i, c: (i, 0)),
          pl.BlockSpec((mb, 128), lambda i, c: (i, c)),
      ],
      out_specs=[
          pl.BlockSpec((mb, 1), lambda i, c: (i, 0)),
          pl.BlockSpec((1, mb, 128), lambda i, c: (c, i, 0)),
      ],
      out_shape=[
          jax.ShapeDtypeStruct((R, 1), jnp.float32),
          jax.ShapeDtypeStruct((2, R, 128), jnp.float32),
      ],
  )(p0, p1, x)


def _mm1_body(a0, a1, w0, w1, d, b, o):
  o[...] = jnp.maximum(
      jnp.dot(a0[...] * d[...], w0[...], preferred_element_type=jnp.float32)
      + jnp.dot(a1[...] * d[...], w1[...],
                preferred_element_type=jnp.float32)
      + b[...], 0.0)


def _mm1(agg1_0, agg1_1, w1_0, w1_1, dis, b1):
  mb, nb = 400, 256
  return pl.pallas_call(
      _mm1_body,
      grid=(N // mb, 1024 // nb),
      in_specs=[
          pl.BlockSpec((mb, 128), lambda i, j: (i, 0)),
          pl.BlockSpec((mb, 128), lambda i, j: (i, 0)),
          pl.BlockSpec((128, nb), lambda i, j: (0, j)),
          pl.BlockSpec((128, nb), lambda i, j: (0, j)),
          pl.BlockSpec((mb, 1), lambda i, j: (i, 0)),
          pl.BlockSpec((1, nb), lambda i, j: (0, j)),
      ],
      out_specs=pl.BlockSpec((mb, nb), lambda i, j: (i, j)),
      out_shape=jax.ShapeDtypeStruct((N, 1024), jnp.float32),
  )(agg1_0, agg1_1, w1_0, w1_1, dis, b1)


def _mm2_body(a, w, d, o):
  r = jnp.dot(a[...], w[...], preferred_element_type=jnp.float32) * d[...]
  o[0] = r[:, :128]
  o[1] = r[:, 128:]


def _mm2(h, w2, dis):
  mb = 400
  return pl.pallas_call(
      _mm2_body,
      grid=(N // mb, 2),
      in_specs=[
          pl.BlockSpec((mb, 1024), lambda i, j: (i, 0)),
          pl.BlockSpec((1024, 256), lambda i, j: (0, j)),
          pl.BlockSpec((mb, 1), lambda i, j: (i, 0)),
      ],
      out_specs=pl.BlockSpec((2, mb, 128), lambda i, j: (j, i, 0)),
      out_shape=jax.ShapeDtypeStruct((4, R, 128), jnp.float32),
  )(h, w2, dis)


def _final_body(a, d, b, o):
  o[:, :128] = a[0] * d[...] + b[0]
  o[:, 128:] = a[1] * d[...] + b[1]


def _final(agg2, dis, b2r):
  mb = 400
  return pl.pallas_call(
      _final_body,
      grid=(N // mb, 2),
      in_specs=[
          pl.BlockSpec((2, mb, 128), lambda i, c: (c, i, 0)),
          pl.BlockSpec((mb, 1), lambda i, c: (i, 0)),
          pl.BlockSpec((2, 1, 128), lambda i, c: (c, 0, 0)),
      ],
      out_specs=pl.BlockSpec((mb, 256), lambda i, c: (i, c)),
      out_shape=jax.ShapeDtypeStruct((N, 512), jnp.float32),
  )(agg2, dis, b2r)


def kernel(x, edge_index, W1, b1, W2, b2):
  src = edge_index[0].astype(jnp.int32)
  dst = edge_index[1].astype(jnp.int32)

  src_t = jnp.pad(src.reshape(NS, EPT), ((0, 0), (0, NB * 128 - EPT)))
  src_t = src_t.reshape(NS, NB, 128)
  dst_t = jnp.pad(dst.reshape(NS, EPT), ((0, 0), (0, NB * 128 - EPT)),
                  constant_values=N).reshape(NS, NB, 128)
  off2 = (R * jnp.arange(2, dtype=jnp.int32))[:, None, None, None]
  off4 = (R * jnp.arange(4, dtype=jnp.int32))[:, None, None, None]
  src2 = src_t[None] + off2
  src4 = src_t[None] + off4

  dst_d = jnp.pad(dst.reshape(NC * NS, EPW),
                  ((0, 0), (0, NBD * 128 - EPW)),
                  constant_values=N).reshape(NC * NS, NBD, 128)

  seq = jnp.arange(R, dtype=jnp.int32).reshape(NS, RP // 128, 128)

  ones_in = jnp.ones((128, 128), jnp.float32)
  zeros_in = jnp.zeros((128, 128), jnp.float32)
  parts = _deg_kernel(dst_d, seq, ones_in, zeros_in).reshape(NC, R, 128)
  dis, vals1 = _predis(parts[0], parts[1], x)
  vals1 = vals1.reshape(2 * R, 128)
  agg1 = _agg2(vals1, src2, dst_t, seq)
  a1r = agg1.reshape(2, R, 128)
  h = _mm1(a1r[0], a1r[1], W1[:128], W1[128:], dis,
           b1.reshape(1, 1024))
  vals2 = _mm2(h, W2, dis).reshape(4 * R, 128)
  agg2 = _agg4(vals2, src4, dst_t, seq)
  return _final(agg2.reshape(4, R, 128), dis, b2.reshape(4, 1, 128))
